# Initial kernel scaffold; baseline (speedup 1.0000x reference)
import jax
import jax.numpy as jnp
from jax.experimental import pallas as pl


def kernel(x, edge_index, edge_attr, batch_labels, label_map, B, W_msg, W_edge, W_self, b_gc, empty_embedding, gru_Wx0, gru_Wh0, gru_bx0, gru_bh0, gru_Wx1, gru_Wh1, gru_bx1, gru_bh1, dec_W, dec_b):
    raise NotImplementedError("write your pallas kernel here")


# speedup vs baseline: 2.9484x; 2.9484x over previous
import functools

import jax
import jax.numpy as jnp
from jax import lax
from jax.experimental import pallas as pl
from jax.experimental.pallas import tpu as pltpu
from jax.experimental.pallas import tpu_sc as plsc

N = 50000
E = 1600000
NP = 50048
S = E // 128
G = 2560
GP = G + 64
T = 10
BQ = 256
H = 128

NC = 2
NS = 16

_SC1_PER_CORE = S // NC
_SC1_Q = _SC1_PER_CORE // NS
_SC1_R = _SC1_PER_CORE - _SC1_Q * NS
_RX = NP // NS
_KG = 13
_NG = _SC1_Q // _KG


def _sc_edge_body(src_hbm, dst_hbm, ea_hbm, x_hbm, z16_hbm, z8_hbm,
                  accx_hbm, acce_hbm,
                  srcs0, dsts0, eas0, srcs1, dsts1, eas1,
                  xr0, xr1,
                  s_src, s_dst, s_ea, s_g0, s_g1, s_sx0, s_sx1, s_se0, s_se1,
                  accx_s, acce_s):
    cid = lax.axis_index("c")
    sid = lax.axis_index("s")

    pltpu.sync_copy(z16_hbm, accx_s.at[pl.ds(sid * _RX, _RX)])
    pltpu.sync_copy(z8_hbm, acce_s.at[pl.ds(sid * _RX, _RX)])
    plsc.subcore_barrier()

    base = cid * _SC1_PER_CORE + sid * _SC1_Q + jnp.minimum(sid, _SC1_R)
    xr = (xr0, xr1)
    s_g = (s_g0, s_g1)
    s_sx = (s_sx0, s_sx1)
    s_se = (s_se0, s_se1)
    slab = ((srcs0, dsts0, eas0), (srcs1, dsts1, eas1))

    def issue_slabs(g, b):
        s0 = base + g * _KG
        for j in range(_KG):
            pltpu.async_copy(src_hbm.at[pl.ds((s0 + j) * 128, 128)],
                             slab[b][0].at[j], s_src)
            pltpu.async_copy(dst_hbm.at[pl.ds((s0 + j) * 128, 128)],
                             slab[b][1].at[j], s_dst)
        pltpu.async_copy(ea_hbm.at[pl.ds(s0, _KG)], slab[b][2], s_ea)

    def wait_slabs(b):
        for j in range(_KG):
            pltpu.make_async_copy(src_hbm.at[pl.ds(0, 128)],
                                  slab[b][0].at[j], s_src).wait()
            pltpu.make_async_copy(dst_hbm.at[pl.ds(0, 128)],
                                  slab[b][1].at[j], s_dst).wait()
        pltpu.make_async_copy(ea_hbm.at[pl.ds(0, _KG)], slab[b][2], s_ea).wait()

    def issue_g(b, j):
        pltpu.async_copy(x_hbm.at[slab[b][0].at[j]], xr[j % 2], s_g[j % 2])

    def wait_g(b, j):
        pltpu.make_async_copy(x_hbm.at[slab[b][0].at[j]], xr[j % 2],
                              s_g[j % 2]).wait()

    def issue_sc(b, j):
        pltpu.async_copy(xr[j % 2], accx_s.at[slab[b][1].at[j]], s_sx[j % 2],
                         add=True)
        pltpu.async_copy(slab[b][2].at[j], acce_s.at[slab[b][1].at[j]],
                         s_se[j % 2], add=True)

    def wait_sc(b, j):
        pltpu.make_async_copy(xr[j % 2], accx_s.at[slab[b][1].at[j]],
                              s_sx[j % 2]).wait()
        pltpu.make_async_copy(slab[b][2].at[j], acce_s.at[slab[b][1].at[j]],
                              s_se[j % 2]).wait()

    def process_group(b):
        for j in range(_KG):
            if j >= 2:
                wait_sc(b, j - 2)
            issue_g(b, j)
            if j >= 1:
                wait_g(b, j - 1)
                issue_sc(b, j - 1)
        wait_sc(b, _KG - 2)
        wait_g(b, _KG - 1)
        issue_sc(b, _KG - 1)

    issue_slabs(0, 0)

    def gp_body(gp, carry):
        @pl.when(gp > 0)
        def _():
            wait_sc(1, _KG - 1)
        wait_slabs(0)
        issue_slabs(2 * gp + 1, 1)
        process_group(0)
        wait_sc(0, _KG - 1)
        wait_slabs(1)

        @pl.when(gp < _NG // 2 - 1)
        def _():
            issue_slabs(2 * gp + 2, 0)
        process_group(1)
        return carry

    lax.fori_loop(0, _NG // 2, gp_body, 0)
    wait_sc(1, _KG - 1)

    @pl.when(sid < _SC1_R)
    def _():
        s = base + _SC1_Q
        pltpu.sync_copy(src_hbm.at[pl.ds(s * 128, 128)], srcs0.at[0])
        pltpu.sync_copy(dst_hbm.at[pl.ds(s * 128, 128)], dsts0.at[0])
        pltpu.sync_copy(ea_hbm.at[pl.ds(s, 1)], eas0.at[pl.ds(0, 1)])
        pltpu.sync_copy(x_hbm.at[srcs0.at[0]], xr0)
        pltpu.sync_copy(xr0, accx_s.at[dsts0.at[0]], add=True)
        pltpu.sync_copy(eas0.at[0], acce_s.at[dsts0.at[0]], add=True)

    plsc.subcore_barrier()
    pltpu.sync_copy(accx_s.at[pl.ds(sid * _RX, _RX)],
                    accx_hbm.at[cid, pl.ds(sid * _RX, _RX)])
    pltpu.sync_copy(acce_s.at[pl.ds(sid * _RX, _RX)],
                    acce_hbm.at[cid, pl.ds(sid * _RX, _RX)])


@functools.partial(
    pl.kernel,
    out_type=[jax.ShapeDtypeStruct((NC, NP, 16), jnp.float32),
              jax.ShapeDtypeStruct((NC, NP, 8), jnp.float32)],
    mesh=plsc.VectorSubcoreMesh(core_axis_name="c", subcore_axis_name="s",
                                num_cores=NC, num_subcores=NS),
    scratch_types=[
        pltpu.VMEM((_KG, 128), jnp.int32),
        pltpu.VMEM((_KG, 128), jnp.int32),
        pltpu.VMEM((_KG, 128, 8), jnp.float32),
        pltpu.VMEM((_KG, 128), jnp.int32),
        pltpu.VMEM((_KG, 128), jnp.int32),
        pltpu.VMEM((_KG, 128, 8), jnp.float32),
        pltpu.VMEM((128, 16), jnp.float32),
        pltpu.VMEM((128, 16), jnp.float32),
        pltpu.SemaphoreType.DMA,
        pltpu.SemaphoreType.DMA,
        pltpu.SemaphoreType.DMA,
        pltpu.SemaphoreType.DMA,
        pltpu.SemaphoreType.DMA,
        pltpu.SemaphoreType.DMA,
        pltpu.SemaphoreType.DMA,
        pltpu.SemaphoreType.DMA,
        pltpu.SemaphoreType.DMA,
        pltpu.VMEM_SHARED((NP, 16), jnp.float32),
        pltpu.VMEM_SHARED((NP, 8), jnp.float32),
    ],
    compiler_params=pltpu.CompilerParams(use_tc_tiling_on_sc=False),
)
def _sc_edge(*args):
    _sc_edge_body(*args)


_S2 = NP // 128
_S2_C0 = (_S2 + 1) // 2
_RG = GP // NS


def _sc_pool_body(lbl_hbm, h_hbm, ones_hbm, z32_hbm, z8_hbm,
                  pool_hbm, cnt_hbm,
                  lbl_v, h_v, one_v, pool_s, cnt_s):
    cid = lax.axis_index("c")
    sid = lax.axis_index("s")

    pltpu.sync_copy(z32_hbm, pool_s.at[pl.ds(sid * _RG, _RG)])
    pltpu.sync_copy(z8_hbm, cnt_s.at[pl.ds(sid * _RG, _RG)])
    pltpu.sync_copy(ones_hbm, one_v)
    plsc.subcore_barrier()

    ncore = jnp.where(cid == 0, _S2_C0, _S2 - _S2_C0)
    q = (_S2_C0) // NS
    r = ncore - q * NS
    nstream = q + jnp.where(sid < r, 1, 0)
    base = cid * _S2_C0 + sid * q + jnp.minimum(sid, r)

    def body(i, carry):
        s = base + i
        pltpu.sync_copy(lbl_hbm.at[s], lbl_v)
        pltpu.sync_copy(h_hbm.at[s], h_v)
        pltpu.sync_copy(h_v, pool_s.at[lbl_v], add=True)
        pltpu.sync_copy(one_v, cnt_s.at[lbl_v], add=True)
        return carry

    lax.fori_loop(0, nstream, body, 0)
    plsc.subcore_barrier()

    pltpu.sync_copy(pool_s.at[pl.ds(sid * _RG, _RG)],
                    pool_hbm.at[cid, pl.ds(sid * _RG, _RG)])
    pltpu.sync_copy(cnt_s.at[pl.ds(sid * _RG, _RG)],
                    cnt_hbm.at[cid, pl.ds(sid * _RG, _RG)])


@functools.partial(
    pl.kernel,
    out_type=[jax.ShapeDtypeStruct((NC, GP, 32), jnp.float32),
              jax.ShapeDtypeStruct((NC, GP, 8), jnp.float32)],
    mesh=plsc.VectorSubcoreMesh(core_axis_name="c", subcore_axis_name="s",
                                num_cores=NC, num_subcores=NS),
    scratch_types=[
        pltpu.VMEM((128,), jnp.int32),
        pltpu.VMEM((128, 32), jnp.float32),
        pltpu.VMEM((128, 8), jnp.float32),
        pltpu.VMEM_SHARED((GP, 32), jnp.float32),
        pltpu.VMEM_SHARED((GP, 8), jnp.float32),
    ],
    compiler_params=pltpu.CompilerParams(use_tc_tiling_on_sc=False),
)
def _sc_pool(*args):
    _sc_pool_body(*args)


_RB = NP // 8


def _tc_h_body(x_ref, ax_ref, ae_ref, wm_ref, we_ref, ws_ref, b_ref, h_ref):
    ax = ax_ref[0] + ax_ref[1]
    ae = ae_ref[0] + ae_ref[1]
    acc = jnp.dot(x_ref[...], ws_ref[...], preferred_element_type=jnp.float32)
    acc = acc + jnp.dot(ax, wm_ref[...], preferred_element_type=jnp.float32)
    acc = acc + jnp.dot(ae, we_ref[...], preferred_element_type=jnp.float32)
    h_ref[...] = jnp.maximum(acc + b_ref[...], 0.0)


def _tc_h(xp, accx, acce, W_msg, W_edge, W_self, b2):
    return pl.pallas_call(
        _tc_h_body,
        grid=(NP // _RB,),
        in_specs=[
            pl.BlockSpec((_RB, 16), lambda i: (i, 0)),
            pl.BlockSpec((NC, _RB, 16), lambda i: (0, i, 0)),
            pl.BlockSpec((NC, _RB, 8), lambda i: (0, i, 0)),
            pl.BlockSpec((16, 32), lambda i: (0, 0)),
            pl.BlockSpec((8, 32), lambda i: (0, 0)),
            pl.BlockSpec((16, 32), lambda i: (0, 0)),
            pl.BlockSpec((1, 32), lambda i: (0, 0)),
        ],
        out_specs=pl.BlockSpec((_RB, 32), lambda i: (i, 0)),
        out_shape=jax.ShapeDtypeStruct((NP, 32), jnp.float32),
    )(xp, accx, acce, W_msg, W_edge, W_self, b2)


def _gru_cell(xt, h, wx, wh, bx, bh):
    gx = jnp.dot(xt, wx, preferred_element_type=jnp.float32) + bx
    gh = jnp.dot(h, wh, preferred_element_type=jnp.float32) + bh
    r = jax.nn.sigmoid(gx[:, 0:H] + gh[:, 0:H])
    z = jax.nn.sigmoid(gx[:, H:2 * H] + gh[:, H:2 * H])
    n = jnp.tanh(gx[:, 2 * H:3 * H] + r * gh[:, 2 * H:3 * H])
    return (1.0 - z) * n + z * h


def _tc_gru_body(pool_ref, cnt_ref,
                 wx0_ref, wh0_ref, bx0_ref, bh0_ref,
                 wx1_ref, wh1_ref, bx1_ref, bh1_ref,
                 dw_ref, db_ref, out_ref, pooled_scr):
    s = pool_ref[0] + pool_ref[1]
    c = cnt_ref[0, :, 0:1] + cnt_ref[1, :, 0:1]
    pooled_scr[...] = (s / jnp.maximum(c, 1.0))[0:G]

    wx0 = wx0_ref[...]
    wh0 = wh0_ref[...]
    bx0 = bx0_ref[...]
    bh0 = bh0_ref[...]
    wx1 = wx1_ref[...]
    wh1 = wh1_ref[...]
    bx1 = bx1_ref[...]
    bh1 = bh1_ref[...]

    def step(t, carry):
        h0, h1 = carry
        xt = pooled_scr[pl.ds(t * BQ, BQ), :]
        h0n = _gru_cell(xt, h0, wx0, wh0, bx0, bh0)
        h1n = _gru_cell(h0n, h1, wx1, wh1, bx1, bh1)
        return (h0n, h1n)

    h0 = jnp.zeros((BQ, H), jnp.float32)
    h1 = jnp.zeros((BQ, H), jnp.float32)
    h0, h1 = lax.fori_loop(0, T, step, (h0, h1))
    out_ref[...] = jnp.dot(h1, dw_ref[...], preferred_element_type=jnp.float32) + db_ref[...]


def _tc_gru(pool, cnt, gru_Wx0, gru_Wh0, bx0, bh0, gru_Wx1, gru_Wh1, bx1, bh1,
            dec_W, db2):
    return pl.pallas_call(
        _tc_gru_body,
        out_shape=jax.ShapeDtypeStruct((BQ, 12), jnp.float32),
        scratch_shapes=[pltpu.VMEM((G, 32), jnp.float32)],
    )(pool, cnt, gru_Wx0, gru_Wh0, bx0, bh0, gru_Wx1, gru_Wh1, bx1, bh1,
      dec_W, db2)


def kernel(x, edge_index, edge_attr, batch_labels, label_map, B,
           W_msg, W_edge, W_self, b_gc, empty_embedding,
           gru_Wx0, gru_Wh0, gru_bx0, gru_bh0,
           gru_Wx1, gru_Wh1, gru_bx1, gru_bh1,
           dec_W, dec_b):
    xp = jnp.pad(x, ((0, NP - N), (0, 0)))
    ea3 = jnp.pad(edge_attr, ((0, 0), (0, 4))).reshape(S, 128, 8)
    z16 = jnp.zeros((_RX, 16), jnp.float32)
    z8 = jnp.zeros((_RX, 8), jnp.float32)

    accx, acce = _sc_edge(edge_index[0].astype(jnp.int32),
                          edge_index[1].astype(jnp.int32), ea3, xp, z16, z8)

    h = _tc_h(xp, accx, acce, W_msg, jnp.pad(W_edge, ((0, 4), (0, 0))),
              W_self, b_gc.reshape(1, 32))

    lblT = (batch_labels % T) * BQ + batch_labels // T
    lblp = jnp.pad(lblT, (0, NP - N), constant_values=G).astype(jnp.int32)
    lbl3 = lblp.reshape(_S2, 128)
    h3 = h.reshape(_S2, 128, 32)
    ones = jnp.ones((128, 8), jnp.float32)
    z32 = jnp.zeros((_RG, 32), jnp.float32)
    z8 = jnp.zeros((_RG, 8), jnp.float32)

    pool, cnt = _sc_pool(lbl3, h3, ones, z32, z8)

    return _tc_gru(pool, cnt,
                   gru_Wx0, gru_Wh0, gru_bx0.reshape(1, 3 * H),
                   gru_bh0.reshape(1, 3 * H),
                   gru_Wx1, gru_Wh1, gru_bx1.reshape(1, 3 * H),
                   gru_bh1.reshape(1, 3 * H),
                   dec_W, dec_b.reshape(1, 12))
